# baseline (speedup 1.0000x reference)
import jax
import jax.numpy as jnp
from jax import lax
from jax.experimental import pallas as pl
from jax.experimental.pallas import tpu as pltpu
from jax.experimental.pallas import tpu_sc as plsc

NC = 2
NS = 16
NW = NC * NS

D = 64
B = 4096 * 200
RPW = B // NW
CHUNK = 128
K = RPW // CHUNK


def _body(x_hbm, table_hbm, out_hbm, idx_v, rows_v, sem):
    wid = lax.axis_index("s") * NC + lax.axis_index("c")
    pltpu.sync_copy(x_hbm.at[wid], idx_v)
    base = wid * RPW

    def step(j, carry):
        pltpu.async_copy(table_hbm.at[idx_v.at[j]], rows_v, sem).wait()
        pltpu.sync_copy(rows_v, out_hbm.at[pl.ds(base + j * CHUNK, CHUNK)])
        return carry

    lax.fori_loop(0, K, step, 0)


@jax.jit
def _lookup(x3, table):
    mesh = plsc.VectorSubcoreMesh(
        core_axis_name="c", subcore_axis_name="s",
        num_cores=NC, num_subcores=NS)
    f = pl.kernel(
        _body,
        out_type=jax.ShapeDtypeStruct((B, D), jnp.float32),
        mesh=mesh,
        scratch_types=[
            pltpu.VMEM((K, CHUNK), jnp.int32),
            pltpu.VMEM((CHUNK, D), jnp.float32),
            pltpu.SemaphoreType.DMA,
        ],
        compiler_params=pltpu.CompilerParams(use_tc_tiling_on_sc=False),
    )
    return f(x3, table)


def kernel(x, table):
    x3 = x.astype(jnp.int32).reshape(NW, K, CHUNK)
    out = _lookup(x3, table)
    return out.reshape(4096, 200, D)


# speedup vs baseline: 1.1152x; 1.1152x over previous
import jax
import jax.numpy as jnp
from jax import lax
from jax.experimental import pallas as pl
from jax.experimental.pallas import tpu as pltpu
from jax.experimental.pallas import tpu_sc as plsc

NC = 2
NS = 16
NW = NC * NS

D = 64
B = 4096 * 200
RPW = B // NW
CHUNK = 128
K = RPW // CHUNK


NBUF = 8
G = K // NBUF


def _body(x_hbm, table_hbm, out_hbm, idx_v, rows, gsems, ssems):
    wid = lax.axis_index("s") * NC + lax.axis_index("c")
    pltpu.sync_copy(x_hbm.at[wid], idx_v)
    base = wid * RPW

    def fire_gather(j, b):
        return pltpu.async_copy(table_hbm.at[idx_v.at[j]], rows.at[b], gsems[b])

    def fire_scatter(j, b):
        pltpu.async_copy(rows.at[b], out_hbm.at[pl.ds(base + j * CHUNK, CHUNK)],
                         ssems[b])

    def wait_scatter(b):
        pltpu.make_async_copy(rows.at[b], out_hbm.at[pl.ds(base, CHUNK)],
                              ssems[b]).wait()

    hs = [fire_gather(b, b) for b in range(NBUF)]
    for b in range(NBUF):
        hs[b].wait()
        fire_scatter(b, b)

    def group(t, carry):
        gh = []
        for b in range(NBUF):
            wait_scatter(b)
            gh.append(fire_gather(t * NBUF + b, b))
        for b in range(NBUF):
            gh[b].wait()
            fire_scatter(t * NBUF + b, b)
        return carry

    lax.fori_loop(1, G, group, 0)
    for b in range(NBUF):
        wait_scatter(b)


@jax.jit
def _lookup(x3, table):
    mesh = plsc.VectorSubcoreMesh(
        core_axis_name="c", subcore_axis_name="s",
        num_cores=NC, num_subcores=NS)
    f = pl.kernel(
        _body,
        out_type=jax.ShapeDtypeStruct((B, D), jnp.float32),
        mesh=mesh,
        scratch_types=[
            pltpu.VMEM((K, CHUNK), jnp.int32),
            pltpu.VMEM((NBUF, CHUNK, D), jnp.float32),
            tuple(pltpu.SemaphoreType.DMA for _ in range(NBUF)),
            tuple(pltpu.SemaphoreType.DMA for _ in range(NBUF)),
        ],
        compiler_params=pltpu.CompilerParams(use_tc_tiling_on_sc=False),
    )
    return f(x3, table)


def kernel(x, table):
    x3 = x.astype(jnp.int32).reshape(NW, K, CHUNK)
    out = _lookup(x3, table)
    return out.reshape(4096, 200, D)


# speedup vs baseline: 1.1161x; 1.0008x over previous
import jax
import jax.numpy as jnp
from jax import lax
from jax.experimental import pallas as pl
from jax.experimental.pallas import tpu as pltpu
from jax.experimental.pallas import tpu_sc as plsc

NC = 2
NS = 16
NW = NC * NS

BATCH = 4096
SEQ = 200
D = 64
BPW = BATCH // NW
C0, C1 = 104, SEQ - 104
NBUF = 8
CPW = 2 * BPW
G = CPW // NBUF


def _body(x_hbm, table_hbm, out_hbm, idx_v, rows, gsems, ssems):
    wid = lax.axis_index("s") * NC + lax.axis_index("c")
    row0 = wid * BPW
    pltpu.sync_copy(x_hbm.at[pl.ds(row0, BPW), :], idx_v)

    def fire_gather(j, b):
        r = j // 2
        c0, n = (0, C0) if b % 2 == 0 else (C0, C1)
        return pltpu.async_copy(
            table_hbm.at[idx_v.at[r, pl.ds(c0, n)]],
            rows.at[b].at[pl.ds(0, n), :], gsems[b])

    def fire_scatter(j, b):
        r = j // 2
        c0, n = (0, C0) if b % 2 == 0 else (C0, C1)
        pltpu.async_copy(rows.at[b].at[pl.ds(0, n), :],
                         out_hbm.at[row0 + r].at[pl.ds(c0, n), :], ssems[b])

    def wait_scatter(b):
        n = C0 if b % 2 == 0 else C1
        pltpu.make_async_copy(rows.at[b].at[pl.ds(0, n), :],
                              out_hbm.at[0].at[pl.ds(0, n), :], ssems[b]).wait()

    hs = [fire_gather(b, b) for b in range(NBUF)]
    for b in range(NBUF):
        hs[b].wait()
        fire_scatter(b, b)

    def group(t, carry):
        gh = []
        for b in range(NBUF):
            wait_scatter(b)
            gh.append(fire_gather(t * NBUF + b, b))
        for b in range(NBUF):
            gh[b].wait()
            fire_scatter(t * NBUF + b, b)
        return carry

    lax.fori_loop(1, G, group, 0)
    for b in range(NBUF):
        wait_scatter(b)


@jax.jit
def _lookup(x, table):
    mesh = plsc.VectorSubcoreMesh(
        core_axis_name="c", subcore_axis_name="s",
        num_cores=NC, num_subcores=NS)
    f = pl.kernel(
        _body,
        out_type=jax.ShapeDtypeStruct((BATCH, SEQ, D), jnp.float32),
        mesh=mesh,
        scratch_types=[
            pltpu.VMEM((BPW, SEQ), jnp.int32),
            pltpu.VMEM((NBUF, C0, D), jnp.float32),
            tuple(pltpu.SemaphoreType.DMA for _ in range(NBUF)),
            tuple(pltpu.SemaphoreType.DMA for _ in range(NBUF)),
        ],
        compiler_params=pltpu.CompilerParams(use_tc_tiling_on_sc=False),
    )
    return f(x, table)


def kernel(x, table):
    return _lookup(x.astype(jnp.int32), table)
